# baseline (speedup 1.0000x reference)
import jax
import jax.numpy as jnp
from jax import lax
from jax.experimental import pallas as pl
from jax.experimental.pallas import tpu as pltpu
from jax.experimental.pallas import tpu_sc as plsc

D = 64
B, T = 4096, 200
V = 1000000
NC, NS = 2, 16
NW = NC * NS
BL = 128
NBUF = 4
SCALE = 8.0
VPB = BL // 16

NBLK = V // 128
TAILW = V - NBLK * 128
NITER_A = 246
NBUF_A = 3

_I16 = lambda: lax.iota(jnp.int32, 16)


def _splat(s):
    return jnp.full((16,), 0, jnp.int32) + s


def _tbody(tT_hbm, t2_hbm, s_bufs, d2_bufs, skew, s_tail, d2_tail,
           gsems, ssems):
    wid = lax.axis_index("s") * NC + lax.axis_index("c")

    def fire(i, j):
        bid = wid + NW * i

        @pl.when(bid < NBLK)
        def _():
            pltpu.async_copy(
                tT_hbm.at[:, pl.ds(bid * 128, 128)], s_bufs[j], gsems[j])

    def wait_gather(j):
        pltpu.make_async_copy(
            tT_hbm.at[:, pl.ds(0, 128)], s_bufs[j], gsems[j]).wait()

    def wait_store(j):
        pltpu.make_async_copy(
            d2_bufs[j], t2_hbm.at[pl.ds(0, 64)], ssems[j]).wait()

    lanes = [_I16() + u0 for u0 in range(0, 128, 16)]

    def transpose_block(src, dst, ncols):
        def stage1(d, c):
            cd = _splat(d)
            for u0 in range(0, ncols, 16):
                v = src[d, pl.ds(u0, 16)]
                plsc.store_scatter(skew, [lanes[u0 // 16], cd], v)
            return c

        lax.fori_loop(0, D, stage1, 0)

        def stage2(p, c):
            for h in range(2):
                ru = _splat(2 * p + h)
                for c0 in range(0, D, 16):
                    v = plsc.load_gather(skew, [ru, lanes[c0 // 16]])
                    dst[p, pl.ds(h * D + c0, 16)] = v
            return c

        lax.fori_loop(0, ncols // 2, stage2, 0)

    fire(0, 0)
    fire(1, 1)

    def block_iter(s, carry):
        for j in range(NBUF_A):
            i = NBUF_A * s + j
            bid = wid + NW * i

            @pl.when(bid < NBLK)
            def _work():
                wait_gather(j)

                @pl.when(i >= NBUF_A)
                def _():
                    wait_store(j)

                transpose_block(s_bufs[j], d2_bufs[j], 128)
                pltpu.async_copy(
                    d2_bufs[j], t2_hbm.at[pl.ds(bid * 64, 64)], ssems[j])

            fire(i + 2, (j + 2) % NBUF_A)
        return carry

    lax.fori_loop(0, NITER_A // NBUF_A, block_iter, 0)

    for j in range(NBUF_A):
        wait_store(j)

    @pl.when(wid == NW - 1)
    def _tail():
        pltpu.sync_copy(tT_hbm.at[:, pl.ds(NBLK * 128, TAILW)], s_tail)
        transpose_block(s_tail, d2_tail, TAILW)
        pltpu.sync_copy(
            d2_tail, t2_hbm.at[pl.ds(NBLK * 64, TAILW // 2)])


def _body(table_hbm, idx_hbm, out_hbm, idx_bufs, pidx_bufs, hoff_bufs,
          pair_bufs, skew, out_bufs, gsems, ssems):
    wid = lax.axis_index("s") * NC + lax.axis_index("c")
    b0 = wid * BL
    lanes = [_I16() + u0 for u0 in range(0, 128, 16)]

    def fire_gathers(t, b):
        pltpu.sync_copy(idx_hbm.at[t, pl.ds(b0, BL)], idx_bufs[b])
        for k in range(VPB):
            v = idx_bufs[b][pl.ds(k * 16, 16)]
            pidx_bufs[b][pl.ds(k * 16, 16)] = lax.shift_right_logical(v, 1)
            hoff_bufs[b][pl.ds(k * 16, 16)] = lax.shift_left(
                lax.bitwise_and(v, 1), 6)
        pltpu.async_copy(table_hbm.at[pidx_bufs[b]], pair_bufs[b], gsems[b])

    def wait_gather(b):
        pltpu.make_async_copy(
            table_hbm.at[pidx_bufs[b]], pair_bufs[b], gsems[b]).wait()

    def wait_store(ob):
        pltpu.make_async_copy(
            out_bufs[ob], out_hbm.at[0, :, 0], ssems[ob]).wait()

    for b in range(NBUF - 1):
        fire_gathers(b, b)

    def chunk_iter(s, carry):
        for b in range(NBUF):
            t = s * NBUF + b
            ob = b % 2
            wait_gather(b)

            @pl.when(t >= 2)
            def _drain_store():
                wait_store(ob)

            def stage1(j, c):
                hvec = hoff_bufs[b][pl.ds(j * 16, 16)]
                for k in range(16):
                    bl = j * 16 + k
                    hk = hvec[k]
                    cb = _splat(bl)
                    for c0 in range(0, D, 16):
                        v = pair_bufs[b][bl, pl.ds(hk + c0, 16)]
                        plsc.store_scatter(skew, [lanes[c0 // 16], cb], v)
                return c

            lax.fori_loop(0, VPB, stage1, 0)

            def stage2(d, c):
                dhi = lax.shift_right_logical(d, 3)
                dlo = lax.bitwise_and(d, 7)
                rd = _splat(d)
                for k in range(VPB):
                    v = plsc.load_gather(skew, [rd, lanes[k]])
                    out_bufs[ob][dhi, dlo, pl.ds(k * 16, 16)] = v * SCALE
                return c

            lax.fori_loop(0, D, stage2, 0)

            pltpu.async_copy(out_bufs[ob], out_hbm.at[t, :, wid], ssems[ob])

            bb = (b + NBUF - 1) % NBUF

            @pl.when(t + NBUF - 1 < T)
            def _prime():
                fire_gathers(t + NBUF - 1, bb)

        return carry

    lax.fori_loop(0, T // NBUF, chunk_iter, 0)

    for ob in range(2):
        wait_store(ob)


@jax.jit
def _emb(tT, idxT):
    mesh = plsc.VectorSubcoreMesh(core_axis_name="c", subcore_axis_name="s")
    t2 = pl.kernel(
        _tbody,
        out_type=jax.ShapeDtypeStruct((V // 2, 128), jnp.float32),
        mesh=mesh,
        compiler_params=pltpu.CompilerParams(
            needs_layout_passes=False, disable_bounds_checks=True),
        scratch_types=[
            [pltpu.VMEM((D, 128), jnp.float32) for _ in range(NBUF_A)],
            [pltpu.VMEM((D, 128), jnp.float32) for _ in range(NBUF_A)],
            pltpu.VMEM((128, 65), jnp.float32),
            pltpu.VMEM((D, TAILW), jnp.float32),
            pltpu.VMEM((TAILW // 2, 128), jnp.float32),
            [pltpu.SemaphoreType.DMA for _ in range(NBUF_A)],
            [pltpu.SemaphoreType.DMA for _ in range(NBUF_A)],
        ],
    )(tT)
    return pl.kernel(
        _body,
        out_type=jax.ShapeDtypeStruct((T, 8, NW, 8, BL), jnp.float32),
        mesh=mesh,
        compiler_params=pltpu.CompilerParams(
            needs_layout_passes=False, disable_bounds_checks=True),
        scratch_types=[
            [pltpu.VMEM((BL,), jnp.int32) for _ in range(NBUF)],
            [pltpu.VMEM((BL,), jnp.int32) for _ in range(NBUF)],
            [pltpu.VMEM((BL,), jnp.int32) for _ in range(NBUF)],
            [pltpu.VMEM((BL, 128), jnp.float32) for _ in range(NBUF)],
            pltpu.VMEM((D, 129), jnp.float32),
            [pltpu.VMEM((8, 8, BL), jnp.float32) for _ in range(2)],
            [pltpu.SemaphoreType.DMA for _ in range(NBUF)],
            [pltpu.SemaphoreType.DMA for _ in range(2)],
        ],
    )(t2, idxT)


def kernel(x, table):
    out5d = _emb(table.T, x.T)
    return out5d.transpose(2, 4, 0, 1, 3).reshape(B, T, D)


# speedup vs baseline: 1.7017x; 1.7017x over previous
import jax
import jax.numpy as jnp
from jax import lax
from jax.experimental import pallas as pl
from jax.experimental.pallas import tpu as pltpu
from jax.experimental.pallas import tpu_sc as plsc

D = 64
B, T = 4096, 200
V = 1000000
NC, NS = 2, 16
NW = NC * NS
BL = 128
NBUF = 4
SCALE = 8.0
VPB = BL // 16

NBLK = V // 128
TAILW = V - NBLK * 128
NITER_A = 246
NBUF_A = 3

_I16 = lambda: lax.iota(jnp.int32, 16)


def _splat(s):
    return jnp.full((16,), 0, jnp.int32) + s


def _tbody(tT_hbm, t2_hbm, s_bufs, d2_bufs, skew, s_tail, d2_tail,
           gsems, ssems):
    wid = lax.axis_index("s") * NC + lax.axis_index("c")

    def fire(i, j):
        bid = wid + NW * i

        @pl.when(bid < NBLK)
        def _():
            pltpu.async_copy(
                tT_hbm.at[:, pl.ds(bid * 128, 128)], s_bufs[j], gsems[j])

    def wait_gather(j):
        pltpu.make_async_copy(
            tT_hbm.at[:, pl.ds(0, 128)], s_bufs[j], gsems[j]).wait()

    def wait_store(j):
        pltpu.make_async_copy(
            d2_bufs[j], t2_hbm.at[pl.ds(0, 64)], ssems[j]).wait()

    lanes = [_I16() + u0 for u0 in range(0, 128, 16)]
    prem65 = [(_I16() + u0) * 65 for u0 in range(0, 128, 16)]

    def transpose_block(src, dst, ncols):
        def stage1(d, c):
            cd = _splat(d)
            for u0 in range(0, ncols, 16):
                v = src[d, pl.ds(u0, 16)]
                plsc.store_scatter(skew, [prem65[u0 // 16] + cd], v)
            return c

        lax.fori_loop(0, D, stage1, 0)

        def stage2(p, c):
            for h in range(2):
                ru = _splat((2 * p + h) * 65)
                for c0 in range(0, D, 16):
                    v = plsc.load_gather(skew, [ru + lanes[c0 // 16]])
                    dst[p, pl.ds(h * D + c0, 16)] = v
            return c

        lax.fori_loop(0, ncols // 2, stage2, 0)

    fire(0, 0)
    fire(1, 1)

    def block_iter(s, carry):
        for j in range(NBUF_A):
            i = NBUF_A * s + j
            bid = wid + NW * i

            @pl.when(bid < NBLK)
            def _work():
                wait_gather(j)

                @pl.when(i >= NBUF_A)
                def _():
                    wait_store(j)

                transpose_block(s_bufs[j], d2_bufs[j], 128)
                pltpu.async_copy(
                    d2_bufs[j], t2_hbm.at[pl.ds(bid * 64, 64)], ssems[j])

            fire(i + 2, (j + 2) % NBUF_A)
        return carry

    lax.fori_loop(0, NITER_A // NBUF_A, block_iter, 0)

    for j in range(NBUF_A):
        wait_store(j)

    @pl.when(wid == NW - 1)
    def _tail():
        pltpu.sync_copy(tT_hbm.at[:, pl.ds(NBLK * 128, TAILW)], s_tail)
        transpose_block(s_tail, d2_tail, TAILW)
        pltpu.sync_copy(
            d2_tail, t2_hbm.at[pl.ds(NBLK * 64, TAILW // 2)])


def _body(table_hbm, idx_hbm, out_hbm, idx_bufs, pidx_bufs, hoff_bufs,
          pair_bufs, skew, out_bufs, gsems, ssems):
    wid = lax.axis_index("s") * NC + lax.axis_index("c")
    b0 = wid * BL
    lanes = [_I16() + u0 for u0 in range(0, 128, 16)]
    prem129 = [(_I16() + c0) * 129 for c0 in range(0, D, 16)]

    def fire_gathers(t, b):
        pltpu.sync_copy(idx_hbm.at[t, pl.ds(b0, BL)], idx_bufs[b])
        for k in range(VPB):
            v = idx_bufs[b][pl.ds(k * 16, 16)]
            pidx_bufs[b][pl.ds(k * 16, 16)] = lax.shift_right_logical(v, 1)
            hoff_bufs[b][pl.ds(k * 16, 16)] = lax.shift_left(
                lax.bitwise_and(v, 1), 6)
        pltpu.async_copy(table_hbm.at[pidx_bufs[b]], pair_bufs[b], gsems[b])

    def wait_gather(b):
        pltpu.make_async_copy(
            table_hbm.at[pidx_bufs[b]], pair_bufs[b], gsems[b]).wait()

    def wait_store(ob):
        pltpu.make_async_copy(
            out_bufs[ob], out_hbm.at[0, :, 0], ssems[ob]).wait()

    for b in range(NBUF - 1):
        fire_gathers(b, b)

    def chunk_iter(s, carry):
        for b in range(NBUF):
            t = s * NBUF + b
            ob = b % 2
            wait_gather(b)

            @pl.when(t >= 2)
            def _drain_store():
                wait_store(ob)

            def stage1(j, c):
                hvec = hoff_bufs[b][pl.ds(j * 16, 16)]
                for k in range(16):
                    bl = j * 16 + k
                    hk = hvec[k]
                    cb = _splat(bl)
                    for c0 in range(0, D, 16):
                        v = pair_bufs[b][bl, pl.ds(hk + c0, 16)]
                        plsc.store_scatter(
                            skew, [prem129[c0 // 16] + cb], v * SCALE)
                return c

            lax.fori_loop(0, VPB, stage1, 0)

            def stage2(d, c):
                dhi = lax.shift_right_logical(d, 3)
                dlo = lax.bitwise_and(d, 7)
                rd = _splat(d * 129)
                for k in range(VPB):
                    v = plsc.load_gather(skew, [rd + lanes[k]])
                    out_bufs[ob][dhi, dlo, pl.ds(k * 16, 16)] = v
                return c

            lax.fori_loop(0, D, stage2, 0)

            pltpu.async_copy(out_bufs[ob], out_hbm.at[t, :, wid], ssems[ob])

            bb = (b + NBUF - 1) % NBUF

            @pl.when(t + NBUF - 1 < T)
            def _prime():
                fire_gathers(t + NBUF - 1, bb)

        return carry

    lax.fori_loop(0, T // NBUF, chunk_iter, 0)

    for ob in range(2):
        wait_store(ob)


@jax.jit
def _emb(tT, idxT):
    mesh = plsc.VectorSubcoreMesh(core_axis_name="c", subcore_axis_name="s")
    t2 = pl.kernel(
        _tbody,
        out_type=jax.ShapeDtypeStruct((V // 2, 128), jnp.float32),
        mesh=mesh,
        compiler_params=pltpu.CompilerParams(
            needs_layout_passes=False, disable_bounds_checks=True),
        scratch_types=[
            [pltpu.VMEM((D, 128), jnp.float32) for _ in range(NBUF_A)],
            [pltpu.VMEM((D, 128), jnp.float32) for _ in range(NBUF_A)],
            pltpu.VMEM((128 * 65,), jnp.float32),
            pltpu.VMEM((D, TAILW), jnp.float32),
            pltpu.VMEM((TAILW // 2, 128), jnp.float32),
            [pltpu.SemaphoreType.DMA for _ in range(NBUF_A)],
            [pltpu.SemaphoreType.DMA for _ in range(NBUF_A)],
        ],
    )(tT)
    return pl.kernel(
        _body,
        out_type=jax.ShapeDtypeStruct((T, 8, NW, 8, BL), jnp.float32),
        mesh=mesh,
        compiler_params=pltpu.CompilerParams(
            needs_layout_passes=False, disable_bounds_checks=True),
        scratch_types=[
            [pltpu.VMEM((BL,), jnp.int32) for _ in range(NBUF)],
            [pltpu.VMEM((BL,), jnp.int32) for _ in range(NBUF)],
            [pltpu.VMEM((BL,), jnp.int32) for _ in range(NBUF)],
            [pltpu.VMEM((BL, 128), jnp.float32) for _ in range(NBUF)],
            pltpu.VMEM((D * 129,), jnp.float32),
            [pltpu.VMEM((8, 8, BL), jnp.float32) for _ in range(2)],
            [pltpu.SemaphoreType.DMA for _ in range(NBUF)],
            [pltpu.SemaphoreType.DMA for _ in range(2)],
        ],
    )(t2, idxT)


def kernel(x, table):
    out5d = _emb(table.T, x.T)
    return out5d.transpose(2, 4, 0, 1, 3).reshape(B, T, D)


# speedup vs baseline: 3.6761x; 2.1603x over previous
import jax
import jax.numpy as jnp
from jax import lax
from jax.experimental import pallas as pl
from jax.experimental.pallas import tpu as pltpu
from jax.experimental.pallas import tpu_sc as plsc

D = 64
B, T = 4096, 200
V = 1000000
NC, NS = 2, 16
NW = NC * NS
BL = 128
NBUF = 4
SCALE = 8.0
VPB = BL // 16

NBLK = V // 128
TAILW = V - NBLK * 128
NITER_A = 246
NBUF_A = 3

_I16 = lambda: lax.iota(jnp.int32, 16)


def _splat(s):
    return jnp.full((16,), 0, jnp.int32) + s


def _tbody(tT_hbm, t2_hbm, s_bufs, d2_bufs, skew, s_tail, d2_tail,
           gsems, ssems):
    wid = lax.axis_index("s") * NC + lax.axis_index("c")

    def fire(i, j):
        bid = wid + NW * i

        @pl.when(bid < NBLK)
        def _():
            pltpu.async_copy(
                tT_hbm.at[:, pl.ds(bid * 128, 128)], s_bufs[j], gsems[j])

    def wait_gather(j):
        pltpu.make_async_copy(
            tT_hbm.at[:, pl.ds(0, 128)], s_bufs[j], gsems[j]).wait()

    def wait_store(j):
        pltpu.make_async_copy(
            d2_bufs[j], t2_hbm.at[pl.ds(0, 64)], ssems[j]).wait()

    lanes = [_I16() + u0 for u0 in range(0, 128, 16)]
    prem65 = [(_I16() + u0) * 65 for u0 in range(0, 128, 16)]

    def transpose_block(src, dst, ncols):
        @plsc.parallel_loop(0, D, unroll=2)
        def stage1(d):
            cd = _splat(d)
            for u0 in range(0, ncols, 16):
                v = src[d, pl.ds(u0, 16)]
                plsc.store_scatter(skew, [prem65[u0 // 16] + cd], v)

        @plsc.parallel_loop(0, ncols // 2, unroll=2)
        def stage2(p):
            for h in range(2):
                ru = _splat((2 * p + h) * 65)
                for c0 in range(0, D, 16):
                    v = plsc.load_gather(skew, [ru + lanes[c0 // 16]])
                    dst[p, pl.ds(h * D + c0, 16)] = v

    fire(0, 0)
    fire(1, 1)

    def block_iter(s, carry):
        for j in range(NBUF_A):
            i = NBUF_A * s + j
            bid = wid + NW * i

            @pl.when(bid < NBLK)
            def _work():
                wait_gather(j)

                @pl.when(i >= NBUF_A)
                def _():
                    wait_store(j)

                transpose_block(s_bufs[j], d2_bufs[j], 128)
                pltpu.async_copy(
                    d2_bufs[j], t2_hbm.at[pl.ds(bid * 64, 64)], ssems[j])

            fire(i + 2, (j + 2) % NBUF_A)
        return carry

    lax.fori_loop(0, NITER_A // NBUF_A, block_iter, 0)

    for j in range(NBUF_A):
        wait_store(j)

    @pl.when(wid == NW - 1)
    def _tail():
        pltpu.sync_copy(tT_hbm.at[:, pl.ds(NBLK * 128, TAILW)], s_tail)
        transpose_block(s_tail, d2_tail, TAILW)
        pltpu.sync_copy(
            d2_tail, t2_hbm.at[pl.ds(NBLK * 64, TAILW // 2)])


def _body(table_hbm, idx_hbm, out_hbm, idx_bufs, pidx_bufs, hoff_bufs,
          pair_bufs, skew, out_bufs, gsems, ssems):
    wid = lax.axis_index("s") * NC + lax.axis_index("c")
    b0 = wid * BL
    lanes = [_I16() + u0 for u0 in range(0, 128, 16)]
    prem129 = [(_I16() + c0) * 129 for c0 in range(0, D, 16)]

    def fire_gathers(t, b):
        pltpu.sync_copy(idx_hbm.at[t, pl.ds(b0, BL)], idx_bufs[b])
        for k in range(VPB):
            v = idx_bufs[b][pl.ds(k * 16, 16)]
            pidx_bufs[b][pl.ds(k * 16, 16)] = lax.shift_right_logical(v, 1)
            hoff_bufs[b][pl.ds(k * 16, 16)] = lax.shift_left(
                lax.bitwise_and(v, 1), 6)
        pltpu.async_copy(table_hbm.at[pidx_bufs[b]], pair_bufs[b], gsems[b])

    def wait_gather(b):
        pltpu.make_async_copy(
            table_hbm.at[pidx_bufs[b]], pair_bufs[b], gsems[b]).wait()

    def wait_store(ob):
        pltpu.make_async_copy(
            out_bufs[ob], out_hbm.at[0, :, 0], ssems[ob]).wait()

    for b in range(NBUF - 1):
        fire_gathers(b, b)

    def chunk_iter(s, carry):
        for b in range(NBUF):
            t = s * NBUF + b
            ob = b % 2
            wait_gather(b)

            @pl.when(t >= 2)
            def _drain_store():
                wait_store(ob)

            @plsc.parallel_loop(0, VPB, unroll=1)
            def stage1(j):
                hvec = hoff_bufs[b][pl.ds(j * 16, 16)]
                for k in range(16):
                    bl = j * 16 + k
                    hk = hvec[k]
                    cb = _splat(bl)
                    for c0 in range(0, D, 16):
                        v = pair_bufs[b][bl, pl.ds(hk + c0, 16)]
                        plsc.store_scatter(
                            skew, [prem129[c0 // 16] + cb], v * SCALE)

            @plsc.parallel_loop(0, D, unroll=2)
            def stage2(d):
                dhi = lax.shift_right_logical(d, 3)
                dlo = lax.bitwise_and(d, 7)
                rd = _splat(d * 129)
                for k in range(VPB):
                    v = plsc.load_gather(skew, [rd + lanes[k]])
                    out_bufs[ob][dhi, dlo, pl.ds(k * 16, 16)] = v

            pltpu.async_copy(out_bufs[ob], out_hbm.at[t, :, wid], ssems[ob])

            bb = (b + NBUF - 1) % NBUF

            @pl.when(t + NBUF - 1 < T)
            def _prime():
                fire_gathers(t + NBUF - 1, bb)

        return carry

    lax.fori_loop(0, T // NBUF, chunk_iter, 0)

    for ob in range(2):
        wait_store(ob)


@jax.jit
def _emb(tT, idxT):
    mesh = plsc.VectorSubcoreMesh(core_axis_name="c", subcore_axis_name="s")
    t2 = pl.kernel(
        _tbody,
        out_type=jax.ShapeDtypeStruct((V // 2, 128), jnp.float32),
        mesh=mesh,
        compiler_params=pltpu.CompilerParams(
            needs_layout_passes=False, disable_bounds_checks=True),
        scratch_types=[
            [pltpu.VMEM((D, 128), jnp.float32) for _ in range(NBUF_A)],
            [pltpu.VMEM((D, 128), jnp.float32) for _ in range(NBUF_A)],
            pltpu.VMEM((128 * 65,), jnp.float32),
            pltpu.VMEM((D, TAILW), jnp.float32),
            pltpu.VMEM((TAILW // 2, 128), jnp.float32),
            [pltpu.SemaphoreType.DMA for _ in range(NBUF_A)],
            [pltpu.SemaphoreType.DMA for _ in range(NBUF_A)],
        ],
    )(tT)
    return pl.kernel(
        _body,
        out_type=jax.ShapeDtypeStruct((T, 8, NW, 8, BL), jnp.float32),
        mesh=mesh,
        compiler_params=pltpu.CompilerParams(
            needs_layout_passes=False, disable_bounds_checks=True),
        scratch_types=[
            [pltpu.VMEM((BL,), jnp.int32) for _ in range(NBUF)],
            [pltpu.VMEM((BL,), jnp.int32) for _ in range(NBUF)],
            [pltpu.VMEM((BL,), jnp.int32) for _ in range(NBUF)],
            [pltpu.VMEM((BL, 128), jnp.float32) for _ in range(NBUF)],
            pltpu.VMEM((D * 129,), jnp.float32),
            [pltpu.VMEM((8, 8, BL), jnp.float32) for _ in range(2)],
            [pltpu.SemaphoreType.DMA for _ in range(NBUF)],
            [pltpu.SemaphoreType.DMA for _ in range(2)],
        ],
    )(t2, idxT)


def kernel(x, table):
    out5d = _emb(table.T, x.T)
    return out5d.transpose(2, 4, 0, 1, 3).reshape(B, T, D)


# speedup vs baseline: 4.3199x; 1.1751x over previous
import jax
import jax.numpy as jnp
from jax import lax
from jax.experimental import pallas as pl
from jax.experimental.pallas import tpu as pltpu
from jax.experimental.pallas import tpu_sc as plsc

D = 64
B, T = 4096, 200
V = 1000000
NC, NS = 2, 16
NW = NC * NS
BL = 128
NBUF = 4
SCALE = 8.0
VPB = BL // 16

NBLK = V // 128
TAILW = V - NBLK * 128
NITER_A = 246
NBUF_A = 3

_I16 = lambda: lax.iota(jnp.int32, 16)


def _splat(s):
    return jnp.full((16,), 0, jnp.int32) + s


def _tbody(tT_hbm, t2_hbm, s_bufs, d2_bufs, skew, s_tail, d2_tail,
           gsems, ssems):
    wid = lax.axis_index("s") * NC + lax.axis_index("c")

    def fire(i, j):
        bid = wid + NW * i

        @pl.when(bid < NBLK)
        def _():
            pltpu.async_copy(
                tT_hbm.at[:, pl.ds(bid * 128, 128)], s_bufs[j], gsems[j])

    def wait_gather(j):
        pltpu.make_async_copy(
            tT_hbm.at[:, pl.ds(0, 128)], s_bufs[j], gsems[j]).wait()

    def wait_store(j):
        pltpu.make_async_copy(
            d2_bufs[j], t2_hbm.at[pl.ds(0, 64)], ssems[j]).wait()

    lanes = [_I16() + u0 for u0 in range(0, 128, 16)]
    prem65 = [(_I16() + u0) * 65 for u0 in range(0, 128, 16)]

    def transpose_block(src, dst, ncols):
        @plsc.parallel_loop(0, D, unroll=2)
        def stage1(d):
            cd = _splat(d)
            for u0 in range(0, ncols, 16):
                v = src[d, pl.ds(u0, 16)]
                plsc.store_scatter(skew, [prem65[u0 // 16] + cd], v)

        @plsc.parallel_loop(0, ncols // 2, unroll=2)
        def stage2(p):
            for h in range(2):
                ru = _splat((2 * p + h) * 65)
                for c0 in range(0, D, 16):
                    v = plsc.load_gather(skew, [ru + lanes[c0 // 16]])
                    dst[p, pl.ds(h * D + c0, 16)] = v

    fire(0, 0)
    fire(1, 1)

    def block_iter(s, carry):
        for j in range(NBUF_A):
            i = NBUF_A * s + j
            bid = wid + NW * i

            @pl.when(bid < NBLK)
            def _work():
                wait_gather(j)

                @pl.when(i >= NBUF_A)
                def _():
                    wait_store(j)

                transpose_block(s_bufs[j], d2_bufs[j], 128)
                pltpu.async_copy(
                    d2_bufs[j], t2_hbm.at[pl.ds(bid * 64, 64)], ssems[j])

            fire(i + 2, (j + 2) % NBUF_A)
        return carry

    lax.fori_loop(0, NITER_A // NBUF_A, block_iter, 0)

    for j in range(NBUF_A):
        wait_store(j)

    @pl.when(wid == NW - 1)
    def _tail():
        pltpu.sync_copy(tT_hbm.at[:, pl.ds(NBLK * 128, TAILW)], s_tail)
        transpose_block(s_tail, d2_tail, TAILW)
        pltpu.sync_copy(
            d2_tail, t2_hbm.at[pl.ds(NBLK * 64, TAILW // 2)])


def _body(table_hbm, idx_hbm, out_hbm, idx_bufs, pidx_bufs, hoff_bufs,
          pair_bufs, skew, out_bufs, gsems, ssems):
    wid = lax.axis_index("s") * NC + lax.axis_index("c")
    b0 = wid * BL
    lanes = [_I16() + u0 for u0 in range(0, 128, 16)]
    prem129 = [(_I16() + c0) * 129 for c0 in range(0, D, 16)]

    def fire_gathers(t, b):
        pltpu.sync_copy(idx_hbm.at[t, pl.ds(b0, BL)], idx_bufs[b])
        for k in range(VPB):
            v = idx_bufs[b][pl.ds(k * 16, 16)]
            pidx_bufs[b][pl.ds(k * 16, 16)] = lax.shift_right_logical(v, 1)
            hoff_bufs[b][pl.ds(k * 16, 16)] = lax.shift_left(
                lax.bitwise_and(v, 1), 6)
        pltpu.async_copy(table_hbm.at[pidx_bufs[b]], pair_bufs[b], gsems[b])

    def wait_gather(b):
        pltpu.make_async_copy(
            table_hbm.at[pidx_bufs[b]], pair_bufs[b], gsems[b]).wait()

    def wait_store(ob):
        pltpu.make_async_copy(
            out_bufs[ob], out_hbm.at[0, :, 0], ssems[ob]).wait()

    for b in range(NBUF - 1):
        fire_gathers(b, b)

    def chunk_iter(s, carry):
        for b in range(NBUF):
            t = s * NBUF + b
            ob = b % 2
            wait_gather(b)

            @pl.when(t >= 2)
            def _drain_store():
                wait_store(ob)

            @plsc.parallel_loop(0, VPB, unroll=2)
            def stage1(j):
                hvec = hoff_bufs[b][pl.ds(j * 16, 16)]
                for k in range(16):
                    bl = j * 16 + k
                    hk = hvec[k]
                    cb = _splat(bl)
                    for c0 in range(0, D, 16):
                        v = pair_bufs[b][bl, pl.ds(hk + c0, 16)]
                        plsc.store_scatter(
                            skew, [prem129[c0 // 16] + cb], v * SCALE)

            @plsc.parallel_loop(0, D, unroll=2)
            def stage2(d):
                dhi = lax.shift_right_logical(d, 3)
                dlo = lax.bitwise_and(d, 7)
                rd = _splat(d * 129)
                for k in range(VPB):
                    v = plsc.load_gather(skew, [rd + lanes[k]])
                    out_bufs[ob][dhi, dlo, pl.ds(k * 16, 16)] = v

            pltpu.async_copy(out_bufs[ob], out_hbm.at[t, :, wid], ssems[ob])

            bb = (b + NBUF - 1) % NBUF

            @pl.when(t + NBUF - 1 < T)
            def _prime():
                fire_gathers(t + NBUF - 1, bb)

        return carry

    lax.fori_loop(0, T // NBUF, chunk_iter, 0)

    for ob in range(2):
        wait_store(ob)


@jax.jit
def _emb(tT, idxT):
    mesh = plsc.VectorSubcoreMesh(core_axis_name="c", subcore_axis_name="s")
    t2 = pl.kernel(
        _tbody,
        out_type=jax.ShapeDtypeStruct((V // 2, 128), jnp.float32),
        mesh=mesh,
        compiler_params=pltpu.CompilerParams(
            needs_layout_passes=False, disable_bounds_checks=True),
        scratch_types=[
            [pltpu.VMEM((D, 128), jnp.float32) for _ in range(NBUF_A)],
            [pltpu.VMEM((D, 128), jnp.float32) for _ in range(NBUF_A)],
            pltpu.VMEM((128 * 65,), jnp.float32),
            pltpu.VMEM((D, TAILW), jnp.float32),
            pltpu.VMEM((TAILW // 2, 128), jnp.float32),
            [pltpu.SemaphoreType.DMA for _ in range(NBUF_A)],
            [pltpu.SemaphoreType.DMA for _ in range(NBUF_A)],
        ],
    )(tT)
    return pl.kernel(
        _body,
        out_type=jax.ShapeDtypeStruct((T, 8, NW, 8, BL), jnp.float32),
        mesh=mesh,
        compiler_params=pltpu.CompilerParams(
            needs_layout_passes=False, disable_bounds_checks=True),
        scratch_types=[
            [pltpu.VMEM((BL,), jnp.int32) for _ in range(NBUF)],
            [pltpu.VMEM((BL,), jnp.int32) for _ in range(NBUF)],
            [pltpu.VMEM((BL,), jnp.int32) for _ in range(NBUF)],
            [pltpu.VMEM((BL, 128), jnp.float32) for _ in range(NBUF)],
            pltpu.VMEM((D * 129,), jnp.float32),
            [pltpu.VMEM((8, 8, BL), jnp.float32) for _ in range(2)],
            [pltpu.SemaphoreType.DMA for _ in range(NBUF)],
            [pltpu.SemaphoreType.DMA for _ in range(2)],
        ],
    )(t2, idxT)


def kernel(x, table):
    out5d = _emb(table.T, x.T)
    return out5d.transpose(2, 4, 0, 1, 3).reshape(B, T, D)
